# baseline (speedup 1.0000x reference)
import functools

import jax
import jax.numpy as jnp
from jax import lax
from jax.experimental import pallas as pl
from jax.experimental.pallas import tpu as pltpu
from jax.experimental.pallas import tpu_sc as plsc

_NB = 50
_H = 512
_W = 512
_NPIX = _H * _W
_CH = 16384
_NCHUNK = _NPIX // _CH
_U = 8
_HBINS = 64
_HSIZE = 16 * _HBINS
_NTILES = 32


def _sc_hist(att_flat, batch):
    ipt = batch // _NTILES
    mesh = plsc.VectorSubcoreMesh(core_axis_name="c", subcore_axis_name="s")

    @functools.partial(
        pl.kernel,
        mesh=mesh,
        out_type=jax.ShapeDtypeStruct((batch * _HSIZE,), jnp.float32),
        compiler_params=pltpu.CompilerParams(needs_layout_passes=False),
        scratch_types=[
            pltpu.VMEM((_CH,), jnp.float32),
            pltpu.VMEM((_CH,), jnp.float32),
            pltpu.VMEM((_HSIZE,), jnp.float32),
            pltpu.SemaphoreType.DMA,
            pltpu.SemaphoreType.DMA,
        ],
    )
    def hist_kernel(att_hbm, his_hbm, buf0, buf1, his_v, sem0, sem1):
        wid = lax.axis_index("s") * 2 + lax.axis_index("c")
        lane = lax.broadcasted_iota(jnp.int32, (16,), 0)
        row = lane * _HBINS
        ones = jnp.ones((16,), jnp.float32)
        zeros = jnp.zeros((16,), jnp.float32)
        bufs = (buf0, buf1)
        sems = (sem0, sem1)

        def process_chunk(buf):
            def step(j, carry):
                base = j * (16 * _U)
                for u in range(_U):
                    v = buf[pl.ds(base + u * 16, 16)]
                    k = jnp.clip((v * float(_NB)).astype(jnp.int32), 0, _NB - 1)
                    plsc.addupdate_scatter(his_v, [row + k], ones)
                return carry
            lax.fori_loop(0, _CH // (16 * _U), step, 0)

        for t in range(ipt):
            im = wid * ipt + t
            pix0 = pl.multiple_of(im * _NPIX, _CH)
            for z in range(_HSIZE // 16):
                his_v[pl.ds(z * 16, 16)] = zeros
            handles = [None, None]
            handles[0] = pltpu.async_copy(
                att_hbm.at[pl.ds(pix0, _CH)], bufs[0], sems[0])
            for c in range(_NCHUNK):
                cur = c % 2
                nxt = (c + 1) % 2
                if c + 1 < _NCHUNK:
                    off = pl.multiple_of(pix0 + (c + 1) * _CH, _CH)
                    handles[nxt] = pltpu.async_copy(
                        att_hbm.at[pl.ds(off, _CH)], bufs[nxt], sems[nxt])
                handles[cur].wait()
                process_chunk(bufs[cur])
            hoff = pl.multiple_of(im * _HSIZE, _HSIZE)
            pltpu.sync_copy(his_v, his_hbm.at[pl.ds(hoff, _HSIZE)])

    return hist_kernel(att_flat)


def _apply_body(att_ref, his_ref, out_ref):
    att = att_ref[0]
    h2 = his_ref[0]
    counts = jnp.sum(h2, axis=0, keepdims=True)
    iota = lax.broadcasted_iota(jnp.int32, (1, _HBINS), 1)
    valid = iota < _NB
    counts = jnp.where(valid, counts, -1.0)

    m = jnp.max(counts)
    ind_max = jnp.min(jnp.where(counts == m, iota, _HBINS))
    masked = jnp.where((iota > ind_max) & valid, counts, -1.0)
    m2 = jnp.max(masked)
    ind_sec = jnp.min(jnp.where(masked == m2, iota, _HBINS))

    thr = ind_sec.astype(jnp.float32) / _NB
    mask = att > thr
    area = jnp.sum(mask.astype(jnp.float32))
    value = jnp.maximum(jnp.sqrt(jnp.sqrt(area)), 1.0)
    inv = 1.0 / value
    supp = jnp.exp(jnp.log(jnp.clip(att, 1e-6, 1.0)) * inv)
    out_ref[0] = jnp.where(mask, supp, att)


def kernel(att_map):
    B = att_map.shape[0]
    x = att_map.reshape(B, _H, _W)
    his = _sc_hist(x.reshape(-1), B).reshape(B, 16, _HBINS)
    out = pl.pallas_call(
        _apply_body,
        grid=(B,),
        in_specs=[
            pl.BlockSpec((1, _H, _W), lambda i: (i, 0, 0)),
            pl.BlockSpec((1, 16, _HBINS), lambda i: (i, 0, 0)),
        ],
        out_specs=pl.BlockSpec((1, _H, _W), lambda i: (i, 0, 0)),
        out_shape=jax.ShapeDtypeStruct((B, _H, _W), jnp.float32),
        compiler_params=pltpu.CompilerParams(
            dimension_semantics=("arbitrary",)),
    )(x, his)
    return jax.lax.stop_gradient(out.reshape(att_map.shape))


# speedup vs baseline: 1.2088x; 1.2088x over previous
import functools

import jax
import jax.numpy as jnp
from jax import lax
from jax.experimental import pallas as pl
from jax.experimental.pallas import tpu as pltpu
from jax.experimental.pallas import tpu_sc as plsc

_NB = 50
_H = 512
_W = 512
_NPIX = _H * _W
_CROWS = 64
_NCHUNK = _H // _CROWS
_NSUB = 8
_HBINS = 64
_HSIZE = 16 * _HBINS
_NTILES = 32
_VPR = _W // 16


def _sc_hist(att_map, batch):
    ipt = batch // _NTILES
    mesh = plsc.VectorSubcoreMesh(core_axis_name="c", subcore_axis_name="s")

    @functools.partial(
        pl.kernel,
        mesh=mesh,
        out_type=jax.ShapeDtypeStruct((batch * _NSUB * _HSIZE,), jnp.float32),
        compiler_params=pltpu.CompilerParams(needs_layout_passes=False),
        scratch_types=[
            pltpu.VMEM((_CROWS, _W), jnp.float32),
            pltpu.VMEM((_CROWS, _W), jnp.float32),
            [pltpu.VMEM((_HSIZE,), jnp.float32) for _ in range(_NSUB)],
            pltpu.SemaphoreType.DMA,
            pltpu.SemaphoreType.DMA,
        ],
    )
    def hist_kernel(att_hbm, his_hbm, buf0, buf1, subs, sem0, sem1):
        wid = lax.axis_index("s") * 2 + lax.axis_index("c")
        lane = lax.broadcasted_iota(jnp.int32, (16,), 0)
        row = lane * _HBINS
        ones = jnp.ones((16,), jnp.float32)
        zeros = jnp.zeros((16,), jnp.float32)
        bufs = (buf0, buf1)
        sems = (sem0, sem1)

        def start_chunk(im, c):
            src = att_hbm.at[im, 0, pl.ds(c * _CROWS, _CROWS), :]
            return pltpu.async_copy(src, bufs[c % 2], sems[c % 2])

        def process_chunk(buf):
            @pl.loop(0, _CROWS)
            def _(r):
                for v in range(_VPR):
                    x = buf[r, pl.ds(v * 16, 16)]
                    k = (x * float(_NB)).astype(jnp.int32)
                    plsc.addupdate_scatter(subs[v % _NSUB], [row + k], ones)

        for t in range(ipt):
            im = wid * ipt + t

            @pl.loop(0, _HSIZE // 16)
            def _(z):
                off = z * 16
                for u in range(_NSUB):
                    subs[u][pl.ds(off, 16)] = zeros

            handles = [None, None]
            handles[0] = start_chunk(im, 0)
            for c in range(_NCHUNK):
                if c + 1 < _NCHUNK:
                    handles[(c + 1) % 2] = start_chunk(im, c + 1)
                handles[c % 2].wait()
                process_chunk(bufs[c % 2])

            for u in range(_NSUB):
                hoff = pl.multiple_of(
                    (im * _NSUB + u) * _HSIZE, _HSIZE)
                pltpu.sync_copy(subs[u], his_hbm.at[pl.ds(hoff, _HSIZE)])

    return hist_kernel(att_map)


def _apply_body(att_ref, his_ref, out_ref):
    att = att_ref[0, 0]
    h2 = his_ref[0]
    counts = jnp.sum(h2, axis=0, keepdims=True)
    iota = lax.broadcasted_iota(jnp.int32, (1, _HBINS), 1)
    valid = iota < _NB
    counts = jnp.where(valid, counts, -1.0)

    m = jnp.max(counts)
    ind_max = jnp.min(jnp.where(counts == m, iota, _HBINS))
    masked = jnp.where((iota > ind_max) & valid, counts, -1.0)
    m2 = jnp.max(masked)
    ind_sec = jnp.min(jnp.where(masked == m2, iota, _HBINS))

    thr = ind_sec.astype(jnp.float32) / _NB
    mask = att > thr
    area = jnp.sum(mask.astype(jnp.float32))
    value = jnp.maximum(jnp.sqrt(jnp.sqrt(area)), 1.0)
    inv = 1.0 / value
    supp = jnp.exp(jnp.log(jnp.clip(att, 1e-6, 1.0)) * inv)
    out_ref[0, 0] = jnp.where(mask, supp, att)


def kernel(att_map):
    B = att_map.shape[0]
    his = _sc_hist(att_map, B).reshape(B, _NSUB * 16, _HBINS)
    out = pl.pallas_call(
        _apply_body,
        grid=(B,),
        in_specs=[
            pl.BlockSpec((1, 1, _H, _W), lambda i: (i, 0, 0, 0)),
            pl.BlockSpec((1, _NSUB * 16, _HBINS), lambda i: (i, 0, 0)),
        ],
        out_specs=pl.BlockSpec((1, 1, _H, _W), lambda i: (i, 0, 0, 0)),
        out_shape=jax.ShapeDtypeStruct((B, 1, _H, _W), jnp.float32),
        compiler_params=pltpu.CompilerParams(
            dimension_semantics=("arbitrary",)),
    )(att_map, his)
    return jax.lax.stop_gradient(out)


# speedup vs baseline: 3.2453x; 2.6848x over previous
import functools

import jax
import jax.numpy as jnp
from jax import lax
from jax.experimental import pallas as pl
from jax.experimental.pallas import tpu as pltpu
from jax.experimental.pallas import tpu_sc as plsc

_NB = 50
_H = 512
_W = 512
_NPIX = _H * _W
_CROWS = 64
_NCHUNK = _H // _CROWS
_NSUB = 8
_HBINS = 64
_HSIZE = 16 * _HBINS
_NTILES = 32
_VPR = _W // 16


def _sc_hist(att_map, batch):
    ipt = batch // _NTILES
    mesh = plsc.VectorSubcoreMesh(core_axis_name="c", subcore_axis_name="s")

    @functools.partial(
        pl.kernel,
        mesh=mesh,
        out_type=jax.ShapeDtypeStruct((batch * _NSUB * _HSIZE,), jnp.float32),
        compiler_params=pltpu.CompilerParams(needs_layout_passes=False),
        scratch_types=[
            pltpu.VMEM((_CROWS, _W), jnp.float32),
            pltpu.VMEM((_CROWS, _W), jnp.float32),
            [pltpu.VMEM((_HSIZE,), jnp.float32) for _ in range(_NSUB)],
            pltpu.SemaphoreType.DMA,
            pltpu.SemaphoreType.DMA,
        ],
    )
    def hist_kernel(att_hbm, his_hbm, buf0, buf1, subs, sem0, sem1):
        wid = lax.axis_index("s") * 2 + lax.axis_index("c")
        lane = lax.broadcasted_iota(jnp.int32, (16,), 0)
        ones = jnp.ones((16,), jnp.float32)
        zeros = jnp.zeros((16,), jnp.float32)
        bufs = (buf0, buf1)
        sems = (sem0, sem1)

        def start_chunk(im, c):
            src = att_hbm.at[im, 0, pl.ds(c * _CROWS, _CROWS), :]
            return pltpu.async_copy(src, bufs[c % 2], sems[c % 2])

        def process_chunk(buf):
            @plsc.parallel_loop(0, _CROWS * 2, unroll=2)
            def _(i):
                r = i >> 1
                base = (i & 1) * (_W // 2)
                xs = [buf[r, pl.ds(base + v * 16, 16)] for v in range(16)]
                fis = [(x * float(_NB)).astype(jnp.int32) * 16 + lane
                       for x in xs]
                for v in range(16):
                    plsc.addupdate_scatter(subs[v % _NSUB], [fis[v]], ones)

        for t in range(ipt):
            im = wid * ipt + t

            @pl.loop(0, _HSIZE // 16)
            def _(z):
                off = z * 16
                for u in range(_NSUB):
                    subs[u][pl.ds(off, 16)] = zeros

            handles = [None, None]
            handles[0] = start_chunk(im, 0)
            for c in range(_NCHUNK):
                if c + 1 < _NCHUNK:
                    handles[(c + 1) % 2] = start_chunk(im, c + 1)
                handles[c % 2].wait()
                process_chunk(bufs[c % 2])

            for u in range(_NSUB):
                hoff = pl.multiple_of(
                    (im * _NSUB + u) * _HSIZE, _HSIZE)
                pltpu.sync_copy(subs[u], his_hbm.at[pl.ds(hoff, _HSIZE)])

    return hist_kernel(att_map)


def _apply_body(att_ref, his_ref, out_ref):
    att = att_ref[0, 0]
    h2 = his_ref[0]
    acc = h2[0:_HBINS]
    for u in range(1, _NSUB):
        acc = acc + h2[u * _HBINS:(u + 1) * _HBINS]
    counts = jnp.sum(acc, axis=1, keepdims=True)
    iota = lax.broadcasted_iota(jnp.int32, (_HBINS, 1), 0)
    valid = iota < _NB
    counts = jnp.where(valid, counts, -1.0)

    m = jnp.max(counts)
    ind_max = jnp.min(jnp.where(counts == m, iota, _HBINS))
    masked = jnp.where((iota > ind_max) & valid, counts, -1.0)
    m2 = jnp.max(masked)
    ind_sec = jnp.min(jnp.where(masked == m2, iota, _HBINS))

    thr = ind_sec.astype(jnp.float32) / _NB
    mask = att > thr
    area = jnp.sum(mask.astype(jnp.float32))
    value = jnp.maximum(jnp.sqrt(jnp.sqrt(area)), 1.0)
    inv = 1.0 / value
    supp = jnp.exp(jnp.log(jnp.clip(att, 1e-6, 1.0)) * inv)
    out_ref[0, 0] = jnp.where(mask, supp, att)


def kernel(att_map):
    B = att_map.shape[0]
    his = _sc_hist(att_map, B).reshape(B, _NSUB * _HBINS, 16)
    out = pl.pallas_call(
        _apply_body,
        grid=(B,),
        in_specs=[
            pl.BlockSpec((1, 1, _H, _W), lambda i: (i, 0, 0, 0)),
            pl.BlockSpec((1, _NSUB * _HBINS, 16), lambda i: (i, 0, 0)),
        ],
        out_specs=pl.BlockSpec((1, 1, _H, _W), lambda i: (i, 0, 0, 0)),
        out_shape=jax.ShapeDtypeStruct((B, 1, _H, _W), jnp.float32),
        compiler_params=pltpu.CompilerParams(
            dimension_semantics=("arbitrary",)),
    )(att_map, his)
    return jax.lax.stop_gradient(out)


# speedup vs baseline: 3.5002x; 1.0785x over previous
import functools

import jax
import jax.numpy as jnp
from jax import lax
from jax.experimental import pallas as pl
from jax.experimental.pallas import tpu as pltpu
from jax.experimental.pallas import tpu_sc as plsc

_NB = 50
_H = 512
_W = 512
_NPIX = _H * _W
_CROWS = 64
_NCHUNK = _H // _CROWS
_NSUB = 8
_HBINS = 64
_HSIZE = 16 * _HBINS
_NTILES = 32
_VPR = _W // 16


def _sc_hist(att_map, batch):
    ipt = batch // _NTILES
    mesh = plsc.VectorSubcoreMesh(core_axis_name="c", subcore_axis_name="s")

    @functools.partial(
        pl.kernel,
        mesh=mesh,
        out_type=jax.ShapeDtypeStruct((batch * _HSIZE,), jnp.float32),
        compiler_params=pltpu.CompilerParams(needs_layout_passes=False),
        scratch_types=[
            pltpu.VMEM((_CROWS, _W), jnp.float32),
            pltpu.VMEM((_CROWS, _W), jnp.float32),
            [pltpu.VMEM((_HSIZE,), jnp.float32) for _ in range(_NSUB)],
            pltpu.SemaphoreType.DMA,
            pltpu.SemaphoreType.DMA,
        ],
    )
    def hist_kernel(att_hbm, his_hbm, buf0, buf1, subs, sem0, sem1):
        wid = lax.axis_index("s") * 2 + lax.axis_index("c")
        lane = lax.broadcasted_iota(jnp.int32, (16,), 0)
        ones = jnp.ones((16,), jnp.float32)
        zeros = jnp.zeros((16,), jnp.float32)
        bufs = (buf0, buf1)
        sems = (sem0, sem1)

        def start_chunk(im, c):
            src = att_hbm.at[im, 0, pl.ds(c * _CROWS, _CROWS), :]
            return pltpu.async_copy(src, bufs[c % 2], sems[c % 2])

        def process_chunk(buf):
            @plsc.parallel_loop(0, _CROWS * 2, unroll=2)
            def _(i):
                r = i >> 1
                base = (i & 1) * (_W // 2)
                xs = [buf[r, pl.ds(base + v * 16, 16)] for v in range(16)]
                fis = [(x * float(_NB)).astype(jnp.int32) * 16 + lane
                       for x in xs]
                for v in range(16):
                    plsc.addupdate_scatter(subs[v % _NSUB], [fis[v]], ones)

        for t in range(ipt):
            im = wid * ipt + t

            @pl.loop(0, _HSIZE // 16)
            def _(z):
                off = z * 16
                for u in range(_NSUB):
                    subs[u][pl.ds(off, 16)] = zeros

            handles = [None, None]
            handles[0] = start_chunk(im, 0)
            for c in range(_NCHUNK):
                if c + 1 < _NCHUNK:
                    handles[(c + 1) % 2] = start_chunk(im, c + 1)
                handles[c % 2].wait()
                process_chunk(bufs[c % 2])

            @pl.loop(0, _HSIZE // 16)
            def _(z):
                off = z * 16
                acc = subs[0][pl.ds(off, 16)]
                for u in range(1, _NSUB):
                    acc = acc + subs[u][pl.ds(off, 16)]
                subs[0][pl.ds(off, 16)] = acc

            hoff = pl.multiple_of(im * _HSIZE, _HSIZE)
            pltpu.sync_copy(subs[0], his_hbm.at[pl.ds(hoff, _HSIZE)])

    return hist_kernel(att_map)


def _apply_body(att_ref, his_ref, out_ref):
    att = att_ref[0, 0]
    h2 = his_ref[0]
    counts = jnp.sum(h2, axis=1, keepdims=True)
    iota = lax.broadcasted_iota(jnp.int32, (_HBINS, 1), 0)
    valid = iota < _NB
    counts = jnp.where(valid, counts, -1.0)

    m = jnp.max(counts)
    ind_max = jnp.min(jnp.where(counts == m, iota, _HBINS))
    masked = jnp.where((iota > ind_max) & valid, counts, -1.0)
    m2 = jnp.max(masked)
    ind_sec = jnp.min(jnp.where(masked == m2, iota, _HBINS))

    thr = ind_sec.astype(jnp.float32) / _NB
    mask = att > thr
    area = jnp.sum(mask.astype(jnp.float32))
    value = jnp.maximum(jnp.sqrt(jnp.sqrt(area)), 1.0)
    inv = 1.0 / value
    supp = jnp.exp(jnp.log(jnp.maximum(att, 1e-6)) * inv)
    out_ref[0, 0] = jnp.where(mask, supp, att)


def kernel(att_map):
    B = att_map.shape[0]
    his = _sc_hist(att_map, B).reshape(B, _HBINS, 16)
    out = pl.pallas_call(
        _apply_body,
        grid=(B,),
        in_specs=[
            pl.BlockSpec((1, 1, _H, _W), lambda i: (i, 0, 0, 0)),
            pl.BlockSpec((1, _HBINS, 16), lambda i: (i, 0, 0)),
        ],
        out_specs=pl.BlockSpec((1, 1, _H, _W), lambda i: (i, 0, 0, 0)),
        out_shape=jax.ShapeDtypeStruct((B, 1, _H, _W), jnp.float32),
        compiler_params=pltpu.CompilerParams(
            dimension_semantics=("arbitrary",)),
    )(att_map, his)
    return jax.lax.stop_gradient(out)


# speedup vs baseline: 4.0240x; 1.1496x over previous
import functools

import jax
import jax.numpy as jnp
from jax import lax
from jax.experimental import pallas as pl
from jax.experimental.pallas import tpu as pltpu
from jax.experimental.pallas import tpu_sc as plsc

_NB = 50
_H = 512
_W = 512
_NPIX = _H * _W
_CROWS = 64
_NCHUNK = _H // _CROWS
_NSUB = 8
_HBINS = 64
_HSIZE = 16 * _HBINS
_NTILES = 32
_VPR = _W // 16


def _sc_hist(att_map, img0, nimg):
    ipt = nimg // _NTILES
    mesh = plsc.VectorSubcoreMesh(core_axis_name="c", subcore_axis_name="s")

    @functools.partial(
        pl.kernel,
        mesh=mesh,
        out_type=jax.ShapeDtypeStruct((nimg * _HSIZE,), jnp.float32),
        compiler_params=pltpu.CompilerParams(needs_layout_passes=False),
        scratch_types=[
            pltpu.VMEM((_CROWS, _W), jnp.float32),
            pltpu.VMEM((_CROWS, _W), jnp.float32),
            [pltpu.VMEM((_HSIZE,), jnp.float32) for _ in range(_NSUB)],
            pltpu.SemaphoreType.DMA,
            pltpu.SemaphoreType.DMA,
        ],
    )
    def hist_kernel(att_hbm, his_hbm, buf0, buf1, subs, sem0, sem1):
        wid = lax.axis_index("s") * 2 + lax.axis_index("c")
        lane = lax.broadcasted_iota(jnp.int32, (16,), 0)
        ones = jnp.ones((16,), jnp.float32)
        zeros = jnp.zeros((16,), jnp.float32)
        bufs = (buf0, buf1)
        sems = (sem0, sem1)

        def start_chunk(im, c):
            src = att_hbm.at[im, 0, pl.ds(c * _CROWS, _CROWS), :]
            return pltpu.async_copy(src, bufs[c % 2], sems[c % 2])

        def process_chunk(buf):
            @plsc.parallel_loop(0, _CROWS * 2, unroll=2)
            def _(i):
                r = i >> 1
                base = (i & 1) * (_W // 2)
                xs = [buf[r, pl.ds(base + v * 16, 16)] for v in range(16)]
                fis = [(x * float(_NB)).astype(jnp.int32) * 16 + lane
                       for x in xs]
                for v in range(16):
                    plsc.addupdate_scatter(subs[v % _NSUB], [fis[v]], ones)

        for t in range(ipt):
            im = img0 + wid * ipt + t

            @pl.loop(0, _HSIZE // 16)
            def _(z):
                off = z * 16
                for u in range(_NSUB):
                    subs[u][pl.ds(off, 16)] = zeros

            handles = [None, None]
            handles[0] = start_chunk(im, 0)
            for c in range(_NCHUNK):
                if c + 1 < _NCHUNK:
                    handles[(c + 1) % 2] = start_chunk(im, c + 1)
                handles[c % 2].wait()
                process_chunk(bufs[c % 2])

            @pl.loop(0, _HSIZE // 16)
            def _(z):
                off = z * 16
                acc = subs[0][pl.ds(off, 16)]
                for u in range(1, _NSUB):
                    acc = acc + subs[u][pl.ds(off, 16)]
                subs[0][pl.ds(off, 16)] = acc

            hoff = pl.multiple_of((im - img0) * _HSIZE, _HSIZE)
            pltpu.sync_copy(subs[0], his_hbm.at[pl.ds(hoff, _HSIZE)])

    return hist_kernel(att_map)


def _apply_next(att_ref, his_ref, prev_ref, out_ref):
    del prev_ref
    _apply_body(att_ref, his_ref, out_ref)


def _apply_body(att_ref, his_ref, out_ref):
    att = att_ref[0, 0]
    h2 = his_ref[0]
    counts = jnp.sum(h2, axis=1, keepdims=True)
    iota = lax.broadcasted_iota(jnp.int32, (_HBINS, 1), 0)
    valid = iota < _NB
    counts = jnp.where(valid, counts, -1.0)

    m = jnp.max(counts)
    ind_max = jnp.min(jnp.where(counts == m, iota, _HBINS))
    masked = jnp.where((iota > ind_max) & valid, counts, -1.0)
    m2 = jnp.max(masked)
    ind_sec = jnp.min(jnp.where(masked == m2, iota, _HBINS))

    thr = ind_sec.astype(jnp.float32) / _NB
    mask = att > thr
    area = jnp.sum(mask.astype(jnp.float32))
    value = jnp.maximum(jnp.sqrt(jnp.sqrt(area)), 1.0)
    inv = 1.0 / value
    supp = jnp.exp(jnp.log(jnp.maximum(att, 1e-6)) * inv)
    out_ref[0, 0] = jnp.where(mask, supp, att)


def kernel(att_map):
    B = att_map.shape[0]
    nslice = 2
    ns = B // nslice
    out = None
    for s in range(nslice):
        img0 = s * ns
        his = _sc_hist(att_map, img0, ns).reshape(ns, _HBINS, 16)
        in_specs = [
            pl.BlockSpec((1, 1, _H, _W),
                         lambda i, o=img0: (i + o, 0, 0, 0)),
            pl.BlockSpec((1, _HBINS, 16), lambda i: (i, 0, 0)),
        ]
        inputs = [att_map, his]
        aliases = {}
        body = _apply_body
        if out is not None:
            in_specs.append(pl.BlockSpec(memory_space=pl.ANY))
            inputs.append(out)
            aliases = {2: 0}
            body = _apply_next
        out = pl.pallas_call(
            body,
            grid=(ns,),
            in_specs=in_specs,
            out_specs=pl.BlockSpec((1, 1, _H, _W),
                                   lambda i, o=img0: (i + o, 0, 0, 0)),
            out_shape=jax.ShapeDtypeStruct((B, 1, _H, _W), jnp.float32),
            input_output_aliases=aliases,
            compiler_params=pltpu.CompilerParams(
                dimension_semantics=("arbitrary",)),
        )(*inputs)
    return jax.lax.stop_gradient(out)


# speedup vs baseline: 5.0006x; 1.2427x over previous
import functools

import jax
import jax.numpy as jnp
from jax import lax
from jax.experimental import pallas as pl
from jax.experimental.pallas import tpu as pltpu
from jax.experimental.pallas import tpu_sc as plsc

_NB = 50
_H = 512
_W = 512
_NPIX = _H * _W
_CROWS = 64
_NCHUNK = _H // _CROWS
_NSUB = 8
_HBINS = 64
_HSIZE = 16 * _HBINS
_NTILES = 32
_VPR = _W // 16


def _sc_hist(att_map, img0, nimg):
    ipt = nimg // _NTILES
    mesh = plsc.VectorSubcoreMesh(core_axis_name="c", subcore_axis_name="s")

    @functools.partial(
        pl.kernel,
        mesh=mesh,
        out_type=jax.ShapeDtypeStruct((nimg * _HSIZE,), jnp.float32),
        compiler_params=pltpu.CompilerParams(needs_layout_passes=False),
        scratch_types=[
            pltpu.VMEM((_CROWS, _W), jnp.float32),
            pltpu.VMEM((_CROWS, _W), jnp.float32),
            [pltpu.VMEM((_HSIZE,), jnp.float32) for _ in range(_NSUB)],
            pltpu.SemaphoreType.DMA,
            pltpu.SemaphoreType.DMA,
        ],
    )
    def hist_kernel(att_hbm, his_hbm, buf0, buf1, subs, sem0, sem1):
        wid = lax.axis_index("s") * 2 + lax.axis_index("c")
        lane = lax.broadcasted_iota(jnp.int32, (16,), 0)
        ones = jnp.ones((16,), jnp.float32)
        zeros = jnp.zeros((16,), jnp.float32)
        bufs = (buf0, buf1)
        sems = (sem0, sem1)

        def start_chunk(im, c):
            src = att_hbm.at[im, 0, pl.ds(c * _CROWS, _CROWS), :]
            return pltpu.async_copy(src, bufs[c % 2], sems[c % 2])

        def process_chunk(buf):
            @plsc.parallel_loop(0, _CROWS * 2, unroll=2)
            def _(i):
                r = i >> 1
                base = (i & 1) * (_W // 2)
                xs = [buf[r, pl.ds(base + v * 16, 16)] for v in range(16)]
                fis = [(x * float(_NB)).astype(jnp.int32) * 16 + lane
                       for x in xs]
                for v in range(16):
                    plsc.addupdate_scatter(subs[v % _NSUB], [fis[v]], ones)

        for t in range(ipt):
            im = img0 + wid * ipt + t

            @pl.loop(0, _HSIZE // 16)
            def _(z):
                off = z * 16
                for u in range(_NSUB):
                    subs[u][pl.ds(off, 16)] = zeros

            handles = [None, None]
            handles[0] = start_chunk(im, 0)
            for c in range(_NCHUNK):
                if c + 1 < _NCHUNK:
                    handles[(c + 1) % 2] = start_chunk(im, c + 1)
                handles[c % 2].wait()
                process_chunk(bufs[c % 2])

            @pl.loop(0, _HSIZE // 16)
            def _(z):
                off = z * 16
                acc = subs[0][pl.ds(off, 16)]
                for u in range(1, _NSUB):
                    acc = acc + subs[u][pl.ds(off, 16)]
                subs[0][pl.ds(off, 16)] = acc

            hoff = pl.multiple_of((im - img0) * _HSIZE, _HSIZE)
            pltpu.sync_copy(subs[0], his_hbm.at[pl.ds(hoff, _HSIZE)])

    return hist_kernel(att_map)


def _apply_next(att_ref, his_ref, prev_ref, out_ref):
    del prev_ref
    _apply_body(att_ref, his_ref, out_ref)


def _apply_body(att_ref, his_ref, out_ref):
    att = att_ref[0, 0]
    h2 = his_ref[0]
    counts = jnp.sum(h2, axis=1, keepdims=True)
    iota = lax.broadcasted_iota(jnp.int32, (_HBINS, 1), 0)
    valid = iota < _NB
    counts = jnp.where(valid, counts, -1.0)

    m = jnp.max(counts)
    ind_max = jnp.min(jnp.where(counts == m, iota, _HBINS))
    masked = jnp.where((iota > ind_max) & valid, counts, -1.0)
    m2 = jnp.max(masked)
    ind_sec = jnp.min(jnp.where(masked == m2, iota, _HBINS))

    thr = ind_sec.astype(jnp.float32) / _NB
    mask = att > thr
    area = jnp.sum(mask.astype(jnp.float32))
    value = jnp.maximum(jnp.sqrt(jnp.sqrt(area)), 1.0)
    inv = 1.0 / value
    supp = jnp.exp(jnp.log(jnp.maximum(att, 1e-6)) * inv)
    out_ref[0, 0] = jnp.where(mask, supp, att)


def kernel(att_map):
    B = att_map.shape[0]
    nslice = 4
    ns = B // nslice
    out = None
    for s in range(nslice):
        img0 = s * ns
        his = _sc_hist(att_map, img0, ns).reshape(ns, _HBINS, 16)
        in_specs = [
            pl.BlockSpec((1, 1, _H, _W),
                         lambda i, o=img0: (i + o, 0, 0, 0)),
            pl.BlockSpec((1, _HBINS, 16), lambda i: (i, 0, 0)),
        ]
        inputs = [att_map, his]
        aliases = {}
        body = _apply_body
        if out is not None:
            in_specs.append(pl.BlockSpec(memory_space=pl.ANY))
            inputs.append(out)
            aliases = {2: 0}
            body = _apply_next
        out = pl.pallas_call(
            body,
            grid=(ns,),
            in_specs=in_specs,
            out_specs=pl.BlockSpec((1, 1, _H, _W),
                                   lambda i, o=img0: (i + o, 0, 0, 0)),
            out_shape=jax.ShapeDtypeStruct((B, 1, _H, _W), jnp.float32),
            input_output_aliases=aliases,
            compiler_params=pltpu.CompilerParams(
                dimension_semantics=("arbitrary",)),
        )(*inputs)
    return jax.lax.stop_gradient(out)
